# Initial kernel scaffold; baseline (speedup 1.0000x reference)
import jax
import jax.numpy as jnp
from jax.experimental import pallas as pl


def kernel(H, W, row_embed, col_embed):
    raise NotImplementedError("write your pallas kernel here")


# speedup vs baseline: 1.1300x; 1.1300x over previous
import functools

import jax
import jax.numpy as jnp
from jax import lax
from jax.experimental import pallas as pl
from jax.experimental.pallas import tpu as pltpu
from jax.experimental.pallas import tpu_sc as plsc

_LANES = 16


@functools.lru_cache(maxsize=None)
def _build(H_s, W_s, D_half):
    info = plsc.get_sparse_core_info()
    NC, NS = info.num_cores, info.num_subcores
    NW = NC * NS
    assert H_s % NW == 0
    HPW = H_s // NW
    D = 2 * D_half
    n_vr = D_half // _LANES

    mesh = plsc.VectorSubcoreMesh(core_axis_name="c", subcore_axis_name="s")

    @functools.partial(
        pl.kernel,
        out_type=jax.ShapeDtypeStruct((H_s * W_s, D), jnp.float32),
        mesh=mesh,
        scratch_types=[
            pltpu.VMEM((W_s, D_half), jnp.float32),
            pltpu.VMEM((HPW, D_half), jnp.float32),
            pltpu.VMEM((W_s, D), jnp.float32),
        ],
    )
    def k(row_hbm, col_hbm, out_hbm, col_v, row_v, buf):
        wid = lax.axis_index("s") * NC + lax.axis_index("c")
        h0 = wid * HPW
        pltpu.sync_copy(col_hbm, col_v)
        pltpu.sync_copy(row_hbm.at[pl.ds(h0, HPW)], row_v)

        def fill_left(w, _):
            for c in range(n_vr):
                buf[w, pl.ds(c * _LANES, _LANES)] = col_v[w, pl.ds(c * _LANES, _LANES)]
            return 0

        lax.fori_loop(0, W_s, fill_left, 0)

        for hl in range(HPW):
            rv = [row_v[hl, pl.ds(c * _LANES, _LANES)] for c in range(n_vr)]

            def fill_right(w, _):
                for c in range(n_vr):
                    buf[w, pl.ds(D_half + c * _LANES, _LANES)] = rv[c]
                return 0

            lax.fori_loop(0, W_s, fill_right, 0)
            pltpu.sync_copy(buf, out_hbm.at[pl.ds((h0 + hl) * W_s, W_s), :])

    return k


def kernel(H, W, row_embed, col_embed):
    H_s, W_s = row_embed.shape[0], col_embed.shape[0]
    D_half = row_embed.shape[1]
    return _build(H_s, W_s, D_half)(row_embed, col_embed)


# speedup vs baseline: 1.2964x; 1.1472x over previous
import functools

import jax
import jax.numpy as jnp
from jax import lax
from jax.experimental import pallas as pl
from jax.experimental.pallas import tpu as pltpu
from jax.experimental.pallas import tpu_sc as plsc

_LANES = 16


@functools.lru_cache(maxsize=None)
def _build(H_s, W_s, D_half):
    info = plsc.get_sparse_core_info()
    NC, NS = info.num_cores, info.num_subcores
    NW = NC * NS
    assert H_s % NW == 0
    HPW = H_s // NW
    D = 2 * D_half
    n_vr = D_half // _LANES

    mesh = plsc.VectorSubcoreMesh(core_axis_name="c", subcore_axis_name="s")

    HW2 = W_s // 2

    @functools.partial(
        pl.kernel,
        out_type=jax.ShapeDtypeStruct((H_s * W_s, D), jnp.float32),
        mesh=mesh,
        scratch_types=[
            pltpu.VMEM((W_s, D_half), jnp.float32),
            pltpu.VMEM((HPW, D_half), jnp.float32),
            pltpu.VMEM((HW2, D), jnp.float32),
            pltpu.VMEM((HW2, D), jnp.float32),
            pltpu.SemaphoreType.DMA,
            pltpu.SemaphoreType.DMA,
        ],
    )
    def k(row_hbm, col_hbm, out_hbm, col_v, row_v, buf_t, buf_b, sem_t, sem_b):
        wid = lax.axis_index("s") * NC + lax.axis_index("c")
        h0 = wid * HPW
        pltpu.sync_copy(col_hbm, col_v)
        pltpu.sync_copy(row_hbm.at[pl.ds(h0, HPW)], row_v)

        def fill_left(w, _):
            for c in range(n_vr):
                buf_t[w, pl.ds(c * _LANES, _LANES)] = col_v[w, pl.ds(c * _LANES, _LANES)]
                buf_b[w, pl.ds(c * _LANES, _LANES)] = col_v[HW2 + w, pl.ds(c * _LANES, _LANES)]
            return 0

        lax.fori_loop(0, HW2, fill_left, 0)

        cp_t = cp_b = None
        for hl in range(HPW):
            rv = [row_v[hl, pl.ds(c * _LANES, _LANES)] for c in range(n_vr)]

            def fill_right(w, _, buf=buf_t, rv=rv):
                for c in range(n_vr):
                    buf[w, pl.ds(D_half + c * _LANES, _LANES)] = rv[c]
                return 0

            if cp_t is not None:
                cp_t.wait()
            lax.fori_loop(0, HW2, fill_right, 0)
            cp_t = pltpu.async_copy(
                buf_t, out_hbm.at[pl.ds((h0 + hl) * W_s, HW2), :], sem_t)

            if cp_b is not None:
                cp_b.wait()
            lax.fori_loop(0, HW2, functools.partial(fill_right, buf=buf_b, rv=rv), 0)
            cp_b = pltpu.async_copy(
                buf_b, out_hbm.at[pl.ds((h0 + hl) * W_s + HW2, HW2), :], sem_b)

        cp_t.wait()
        cp_b.wait()

    return k


def kernel(H, W, row_embed, col_embed):
    H_s, W_s = row_embed.shape[0], col_embed.shape[0]
    D_half = row_embed.shape[1]
    return _build(H_s, W_s, D_half)(row_embed, col_embed)
